# baseline (speedup 1.0000x reference)
import functools
import math

import jax
import jax.numpy as jnp
from jax import lax
from jax.experimental import pallas as pl
from jax.experimental.pallas import tpu as pltpu
from jax.experimental.pallas import tpu_sc as plsc

_RAD = 15.0 * (math.pi / 180.0)

_NC, _NS = 2, 16
_NW = _NC * _NS


@functools.lru_cache(maxsize=None)
def _build_gather(B, V, D):
    assert B % _NW == 0, (B, _NW)
    b_per_w = B // _NW
    mesh = plsc.VectorSubcoreMesh(core_axis_name="c", subcore_axis_name="s")

    @functools.partial(
        pl.kernel,
        mesh=mesh,
        out_type=jax.ShapeDtypeStruct((B, D), jnp.float32),
        scratch_types=[
            pltpu.VMEM((b_per_w,), jnp.int32),
            pltpu.SMEM((b_per_w,), jnp.int32),
            pltpu.SemaphoreType.DMA,
        ],
        compiler_params=pltpu.CompilerParams(use_tc_tiling_on_sc=True),
    )
    def gather_rows(idx_hbm, table_hbm, out_hbm, idx_v, idx_s, sem):
        wid = lax.axis_index("s") * _NC + lax.axis_index("c")
        base = wid * b_per_w
        pltpu.sync_copy(idx_hbm.at[pl.ds(base, b_per_w)], idx_v)
        for g in range(b_per_w // 16):
            vec = idx_v[pl.ds(g * 16, 16)]
            for k in range(16):
                idx_s[g * 16 + k] = vec[k]

        def enqueue(i, carry):
            row = idx_s[i]
            pltpu.async_copy(
                table_hbm.at[pl.ds(row, 1)],
                out_hbm.at[pl.ds(base + i, 1)],
                sem,
            )
            return carry

        lax.fori_loop(0, b_per_w, enqueue, 0)

        def drain(i, carry):
            pltpu.make_async_copy(
                table_hbm.at[pl.ds(0, 1)],
                out_hbm.at[pl.ds(base, 1)],
                sem,
            ).wait()
            return carry

        lax.fori_loop(0, b_per_w, drain, 0)

    return gather_rows


@functools.lru_cache(maxsize=None)
def _build_tanh(B, D, rows_per_block):
    grid = B // rows_per_block

    def tanh_body(res_ref, x_ref, o_ref):
        o_ref[...] = jnp.tanh(x_ref[...]) * _RAD + res_ref[0]

    return pl.pallas_call(
        tanh_body,
        grid=(grid,),
        in_specs=[
            pl.BlockSpec(memory_space=pltpu.SMEM),
            pl.BlockSpec((rows_per_block, D), lambda i: (i, 0)),
        ],
        out_specs=pl.BlockSpec((rows_per_block, D), lambda i: (i, 0)),
        out_shape=jax.ShapeDtypeStruct((B, D), jnp.float32),
    )


def kernel(sample_index, bones_rotations_weight, num_bones):
    B = sample_index.shape[0]
    V, D = bones_rotations_weight.shape
    nb = D // 3
    idx = sample_index.astype(jnp.int32)
    res = jnp.reshape(jnp.asarray(num_bones, jnp.float32) - jnp.float32(nb), (1,))
    gathered = _build_gather(B, V, D)(idx, bones_rotations_weight)
    out = _build_tanh(B, D, 2048)(res, gathered)
    return out.reshape(B, nb, 3)


# speedup vs baseline: 1.0924x; 1.0924x over previous
import functools
import math

import jax
import jax.numpy as jnp
from jax import lax
from jax.experimental import pallas as pl
from jax.experimental.pallas import tpu as pltpu
from jax.experimental.pallas import tpu_sc as plsc

_RAD = 15.0 * (math.pi / 180.0)

_NC, _NS = 2, 16
_NW = _NC * _NS
_CHUNK = 128
_DPAD = 128


@functools.lru_cache(maxsize=None)
def _build_gather(B, V):
    assert B % (_NW * _CHUNK) == 0, (B, _NW, _CHUNK)
    b_per_w = B // _NW
    n_chunks = b_per_w // _CHUNK
    mesh = plsc.VectorSubcoreMesh(core_axis_name="c", subcore_axis_name="s")

    @functools.partial(
        pl.kernel,
        mesh=mesh,
        out_type=jax.ShapeDtypeStruct((B, _DPAD), jnp.float32),
        scratch_types=(
            [pltpu.VMEM((_CHUNK,), jnp.int32)] * n_chunks
            + [
                pltpu.VMEM((b_per_w, _DPAD), jnp.float32),
                pltpu.SemaphoreType.DMA,
            ]
        ),
        compiler_params=pltpu.CompilerParams(use_tc_tiling_on_sc=True),
    )
    def gather_rows(idx_hbm, table_hbm, out_hbm, *rest):
        idx_refs = rest[:n_chunks]
        rows_v, sem = rest[n_chunks:]
        wid = lax.axis_index("s") * _NC + lax.axis_index("c")
        base = wid * b_per_w
        for j in range(n_chunks):
            pltpu.sync_copy(idx_hbm.at[wid * n_chunks + j], idx_refs[j])
        copies = [
            pltpu.async_copy(
                table_hbm.at[idx_refs[j]],
                rows_v.at[pl.ds(j * _CHUNK, _CHUNK)],
                sem,
            )
            for j in range(n_chunks)
        ]
        for c in copies:
            c.wait()
        pltpu.sync_copy(rows_v, out_hbm.at[pl.ds(base, b_per_w)])

    return gather_rows


@functools.lru_cache(maxsize=None)
def _build_tanh(B, D, rows_per_block):
    grid = B // rows_per_block

    def tanh_body(res_ref, x_ref, o_ref):
        o_ref[...] = jnp.tanh(x_ref[:, :D]) * _RAD + res_ref[0]

    return pl.pallas_call(
        tanh_body,
        grid=(grid,),
        in_specs=[
            pl.BlockSpec(memory_space=pltpu.SMEM),
            pl.BlockSpec((rows_per_block, _DPAD), lambda i: (i, 0)),
        ],
        out_specs=pl.BlockSpec((rows_per_block, D), lambda i: (i, 0)),
        out_shape=jax.ShapeDtypeStruct((B, D), jnp.float32),
    )


def kernel(sample_index, bones_rotations_weight, num_bones):
    B = sample_index.shape[0]
    V, D = bones_rotations_weight.shape
    nb = D // 3
    idx = sample_index.astype(jnp.int32).reshape(B // _CHUNK, _CHUNK)
    table128 = jnp.pad(bones_rotations_weight, ((0, 0), (0, _DPAD - D)))
    res = jnp.reshape(jnp.asarray(num_bones, jnp.float32) - jnp.float32(nb), (1,))
    gathered = _build_gather(B, V)(idx, table128)
    out = _build_tanh(B, D, 2048)(res, gathered)
    return out.reshape(B, nb, 3)
